# Initial kernel scaffold; baseline (speedup 1.0000x reference)
import functools

import jax
import jax.numpy as jnp
from jax import lax
from jax.experimental import pallas as pl
from jax.experimental.pallas import tpu as pltpu
from jax.experimental.pallas import tpu_sc as plsc

_INFO = plsc.get_sparse_core_info()
_NC = _INFO.num_cores
_NS = _INFO.num_subcores
_NW = _NC * _NS
_CHUNK = 128


@functools.partial(jax.jit, static_argnames=("n_chunks", "dim"))
def _sc_gather(idx_grp, weight, n_chunks, dim):
    b_per_w = n_chunks * _CHUNK
    mesh = plsc.VectorSubcoreMesh(core_axis_name="c", subcore_axis_name="s")

    @functools.partial(
        pl.kernel,
        out_type=jax.ShapeDtypeStruct((_NW, b_per_w, dim), jnp.float32),
        mesh=mesh,
        scratch_types=[
            pltpu.VMEM((n_chunks, _CHUNK), jnp.int32),
            pltpu.VMEM((_CHUNK, dim), jnp.float32),
            pltpu.VMEM((_CHUNK, dim), jnp.float32),
            pltpu.SemaphoreType.DMA,
        ],
    )
    def k(idx_hbm, table_hbm, out_hbm, idx_v, rows0, rows1, gsem):
        wid = lax.axis_index("s") * _NC + lax.axis_index("c")
        pltpu.sync_copy(idx_hbm.at[wid], idx_v)

        pltpu.async_copy(table_hbm.at[idx_v.at[0]], rows0, gsem)

        def outer(g, carry):
            j0 = 2 * g
            j1 = 2 * g + 1
            pltpu.async_copy(table_hbm.at[idx_v.at[j1]], rows1, gsem)
            pltpu.make_async_copy(table_hbm.at[idx_v.at[j0]], rows0, gsem).wait()
            pltpu.sync_copy(rows0, out_hbm.at[wid, pl.ds(j0 * _CHUNK, _CHUNK)])

            @pl.when(j1 + 1 < n_chunks)
            def _():
                pltpu.async_copy(table_hbm.at[idx_v.at[j1 + 1]], rows0, gsem)

            pltpu.make_async_copy(table_hbm.at[idx_v.at[j1]], rows1, gsem).wait()
            pltpu.sync_copy(rows1, out_hbm.at[wid, pl.ds(j1 * _CHUNK, _CHUNK)])
            return carry

        lax.fori_loop(0, n_chunks // 2, outer, 0)

    return k(idx_grp, weight)


def kernel(input_ids, weight):
    b, s = input_ids.shape
    dim = weight.shape[1]
    total = b * s
    assert total % (_NW * _CHUNK) == 0 and (total // (_NW * _CHUNK)) % 2 == 0
    n_chunks = total // (_NW * _CHUNK)
    idx_grp = input_ids.reshape(_NW, n_chunks, _CHUNK).astype(jnp.int32)
    out = _sc_gather(idx_grp, weight, n_chunks, dim)
    return out.reshape(b, s, dim)


# speedup vs baseline: 4.5332x; 4.5332x over previous
import functools

import jax
import jax.numpy as jnp
from jax import lax
from jax.experimental import pallas as pl
from jax.experimental.pallas import tpu as pltpu
from jax.experimental.pallas import tpu_sc as plsc

_INFO = plsc.get_sparse_core_info()
_NC = _INFO.num_cores
_NS = _INFO.num_subcores
_NW = _NC * _NS
_CHUNK = 128


@functools.partial(jax.jit, static_argnames=("n_chunks", "dim"))
def _sc_gather(idx_grp, weight, n_chunks, dim):
    b_per_w = n_chunks * _CHUNK
    mesh = plsc.VectorSubcoreMesh(core_axis_name="c", subcore_axis_name="s")

    @functools.partial(
        pl.kernel,
        out_type=jax.ShapeDtypeStruct((_NW, b_per_w, dim), jnp.float32),
        mesh=mesh,
        scratch_types=[
            pltpu.VMEM((n_chunks, _CHUNK), jnp.int32),
            pltpu.VMEM((_CHUNK, dim), jnp.float32),
            pltpu.VMEM((_CHUNK, dim), jnp.float32),
            pltpu.SemaphoreType.DMA,
        ],
        compiler_params=pltpu.CompilerParams(use_tc_tiling_on_sc=False),
    )
    def k(idx_hbm, table_hbm, out_hbm, idx_v, rows0, rows1, gsem):
        wid = lax.axis_index("s") * _NC + lax.axis_index("c")
        pltpu.sync_copy(idx_hbm.at[wid], idx_v)

        pltpu.async_copy(table_hbm.at[idx_v.at[0]], rows0, gsem)

        def outer(g, carry):
            j0 = 2 * g
            j1 = 2 * g + 1
            pltpu.async_copy(table_hbm.at[idx_v.at[j1]], rows1, gsem)
            pltpu.make_async_copy(table_hbm.at[idx_v.at[j0]], rows0, gsem).wait()
            pltpu.sync_copy(rows0, out_hbm.at[wid, pl.ds(j0 * _CHUNK, _CHUNK)])

            @pl.when(j1 + 1 < n_chunks)
            def _():
                pltpu.async_copy(table_hbm.at[idx_v.at[j1 + 1]], rows0, gsem)

            pltpu.make_async_copy(table_hbm.at[idx_v.at[j1]], rows1, gsem).wait()
            pltpu.sync_copy(rows1, out_hbm.at[wid, pl.ds(j1 * _CHUNK, _CHUNK)])
            return carry

        lax.fori_loop(0, n_chunks // 2, outer, 0)

    return k(idx_grp, weight)


def kernel(input_ids, weight):
    b, s = input_ids.shape
    dim = weight.shape[1]
    total = b * s
    assert total % (_NW * _CHUNK) == 0 and (total // (_NW * _CHUNK)) % 2 == 0
    n_chunks = total // (_NW * _CHUNK)
    idx_grp = input_ids.reshape(_NW, n_chunks, _CHUNK).astype(jnp.int32)
    out = _sc_gather(idx_grp, weight, n_chunks, dim)
    return out.reshape(b, s, dim)


# speedup vs baseline: 4.6826x; 1.0330x over previous
import functools

import jax
import jax.numpy as jnp
from jax import lax
from jax.experimental import pallas as pl
from jax.experimental.pallas import tpu as pltpu
from jax.experimental.pallas import tpu_sc as plsc

_INFO = plsc.get_sparse_core_info()
_NC = _INFO.num_cores
_NS = _INFO.num_subcores
_NW = _NC * _NS
_CHUNK = 128


@functools.partial(jax.jit, static_argnames=("n_chunks", "dim"))
def _sc_gather(idx_grp, weight, n_chunks, dim):
    b_per_w = n_chunks * _CHUNK
    mesh = plsc.VectorSubcoreMesh(core_axis_name="c", subcore_axis_name="s")

    nbuf = 10
    pref = 4
    assert n_chunks % nbuf == 0 and n_chunks >= nbuf

    @functools.partial(
        pl.kernel,
        out_type=jax.ShapeDtypeStruct((_NW, b_per_w, dim), jnp.float32),
        mesh=mesh,
        scratch_types=[
            pltpu.VMEM((n_chunks, _CHUNK), jnp.int32),
            pltpu.VMEM((nbuf, _CHUNK, dim), jnp.float32),
            pltpu.SemaphoreType.DMA,
            pltpu.SemaphoreType.DMA,
        ],
        compiler_params=pltpu.CompilerParams(use_tc_tiling_on_sc=False),
    )
    def k(idx_hbm, table_hbm, out_hbm, idx_v, rows_v, gsem, ssem):
        wid = lax.axis_index("s") * _NC + lax.axis_index("c")
        pltpu.sync_copy(idx_hbm.at[wid], idx_v)

        def gather(j, buf):
            pltpu.async_copy(table_hbm.at[idx_v.at[j]], rows_v.at[buf], gsem)

        def store_desc(j, buf):
            return pltpu.make_async_copy(
                rows_v.at[buf], out_hbm.at[wid, pl.ds(j * _CHUNK, _CHUNK)], ssem
            )

        for m in range(pref):
            gather(m, m)

        def outer(g, carry):
            for i in range(nbuf):
                j = nbuf * g + i

                @pl.when(j - (nbuf - pref) >= 0)
                def _(i=i, j=j):
                    store_desc(j - (nbuf - pref), (i + pref) % nbuf).wait()

                @pl.when(j + pref < n_chunks)
                def _(i=i, j=j):
                    gather(j + pref, (i + pref) % nbuf)

                pltpu.make_async_copy(
                    table_hbm.at[idx_v.at[j]], rows_v.at[i], gsem
                ).wait()
                store_desc(j, i).start()
            return carry

        lax.fori_loop(0, n_chunks // nbuf, outer, 0)
        for j in range(n_chunks - (nbuf - pref), n_chunks):
            store_desc(j, j % nbuf).wait()

    return k(idx_grp, weight)


def kernel(input_ids, weight):
    b, s = input_ids.shape
    dim = weight.shape[1]
    total = b * s
    assert total % (_NW * _CHUNK) == 0 and (total // (_NW * _CHUNK)) % 2 == 0
    n_chunks = total // (_NW * _CHUNK)
    idx_grp = input_ids.reshape(_NW, n_chunks, _CHUNK).astype(jnp.int32)
    out = _sc_gather(idx_grp, weight, n_chunks, dim)
    return out.reshape(b, s, dim)


# speedup vs baseline: 4.6917x; 1.0019x over previous
import functools

import jax
import jax.numpy as jnp
from jax import lax
from jax.experimental import pallas as pl
from jax.experimental.pallas import tpu as pltpu
from jax.experimental.pallas import tpu_sc as plsc

_INFO = plsc.get_sparse_core_info()
_NC = _INFO.num_cores
_NS = _INFO.num_subcores
_NW = _NC * _NS


@functools.partial(jax.jit, static_argnames=("b", "s", "dim"))
def _sc_gather(idx2, weight, b, s, dim):
    chunk = 2 * s
    n_chunks = b // (2 * _NW)
    mesh = plsc.VectorSubcoreMesh(core_axis_name="c", subcore_axis_name="s")

    nbuf = 8
    pref = 3
    assert n_chunks % nbuf == 0 and n_chunks >= nbuf

    @functools.partial(
        pl.kernel,
        out_type=jax.ShapeDtypeStruct((b, s, dim), jnp.float32),
        mesh=mesh,
        scratch_types=[
            pltpu.VMEM((n_chunks, chunk), jnp.int32),
            pltpu.VMEM((nbuf, chunk, dim), jnp.float32),
            pltpu.SemaphoreType.DMA,
            pltpu.SemaphoreType.DMA,
        ],
        compiler_params=pltpu.CompilerParams(use_tc_tiling_on_sc=False),
    )
    def k(idx_hbm, table_hbm, out_hbm, idx_v, rows_v, gsem, ssem):
        wid = lax.axis_index("s") * _NC + lax.axis_index("c")
        batch0 = wid * (2 * n_chunks)
        pltpu.sync_copy(idx_hbm.at[pl.ds(wid * n_chunks, n_chunks)], idx_v)

        def gather(j, buf):
            pltpu.async_copy(table_hbm.at[idx_v.at[j]], rows_v.at[buf], gsem)

        def store_desc(j, buf, half):
            return pltpu.make_async_copy(
                rows_v.at[buf, pl.ds(half * s, s)],
                out_hbm.at[batch0 + 2 * j + half],
                ssem,
            )

        for m in range(pref):
            gather(m, m)

        def outer(g, carry):
            for i in range(nbuf):
                j = nbuf * g + i

                @pl.when(j - (nbuf - pref) >= 0)
                def _(i=i, j=j):
                    store_desc(j - (nbuf - pref), (i + pref) % nbuf, 0).wait()
                    store_desc(j - (nbuf - pref), (i + pref) % nbuf, 1).wait()

                @pl.when(j + pref < n_chunks)
                def _(i=i, j=j):
                    gather(j + pref, (i + pref) % nbuf)

                pltpu.make_async_copy(
                    table_hbm.at[idx_v.at[j]], rows_v.at[i], gsem
                ).wait()
                store_desc(j, i, 0).start()
                store_desc(j, i, 1).start()
            return carry

        lax.fori_loop(0, n_chunks // nbuf, outer, 0)
        for j in range(n_chunks - (nbuf - pref), n_chunks):
            store_desc(j, j % nbuf, 0).wait()
            store_desc(j, j % nbuf, 1).wait()

    return k(idx2, weight)


def kernel(input_ids, weight):
    b, s = input_ids.shape
    dim = weight.shape[1]
    assert b % (2 * _NW) == 0
    idx2 = input_ids.reshape(b // 2, 2 * s).astype(jnp.int32)
    return _sc_gather(idx2, weight, b, s, dim)
